# baseline (device time: 76190 ns/iter reference)
import jax
import jax.numpy as jnp
from jax import lax
from jax.experimental import pallas as pl
from jax.experimental.pallas import tpu as pltpu

N_DEV = 8
B, SQ, D = 4, 256, 1024
HQ_LOC, DH = 8, 128
SKV = 1024
SCALE = 0.08838834764831843
BF = jnp.bfloat16
F32 = jnp.float32

COLS = ((0, 384), (384, 768), (768, 1024))
ORDERS = ((0, 1, 2), (1, 2, 0), (2, 0, 1))


def kernel(x, Wq, Wo, K_ext, V_ext):
    def body(x_ref, wq_ref, wo_ref, k_hbm, v_hbm, out_ref,
             w0_ref, w1_ref, w2_ref, r0_ref, r1_ref, r2_ref,
             q_ref, kv_ref, copy_sem, send_sems, recv_sems):
        a_ref = q_ref
        me = lax.axis_index("i")
        s = lax.rem(me, 4)
        z = lax.div(me, 4)
        xb = lax.rem(s + lax.div(s, 2), 2)
        yb = lax.div(s, 2)
        px = z * 4 + s + 1 - 2 * lax.rem(s, 2)
        py = z * 4 + 3 - s
        pz = (1 - z) * 4 + s
        partner_ax = (px, py, pz)
        mybit = (xb, yb, z)
        w_refs = (w0_ref, w1_ref, w2_ref)
        r_refs = (r0_ref, r1_ref, r2_ref)

        geom = []
        for p in range(3):
            o = ORDERS[p]
            geom.append((mybit[o[0]], mybit[o[1]], mybit[o[2]]))

        RSTART = (0, 4, 6)

        def rs_rdma(p, t):
            b0, b1, b2 = geom[p]
            if t == 0:
                keep, send, n = 4 * b0, 4 * (1 - b0), 4
            elif t == 1:
                keep, send, n = 4 * b0 + 2 * b1, 4 * b0 + 2 * (1 - b1), 2
            else:
                keep = 4 * b0 + 2 * b1 + b2
                send, n = 4 * b0 + 2 * b1 + (1 - b2), 1
            rdma = pltpu.make_async_remote_copy(
                src_ref=w_refs[p].at[pl.ds(send, n)],
                dst_ref=r_refs[p].at[pl.ds(RSTART[t], n)],
                send_sem=send_sems.at[p, t],
                recv_sem=recv_sems.at[p, t],
                device_id=(partner_ax[ORDERS[p][t]],),
                device_id_type=pl.DeviceIdType.MESH)
            return rdma, keep, n

        def rs_finish(p, t, pending):
            rdma, keep, n = pending
            rdma.wait()
            w_refs[p][pl.ds(keep, n)] = (
                w_refs[p][pl.ds(keep, n)]
                + r_refs[p][pl.ds(RSTART[t], n)])

        def ag_rdma(p, u):
            b0, b1, b2 = geom[p]
            if u == 0:
                start, n = 4 * b0 + 2 * b1 + b2, 1
            elif u == 1:
                start, n = 4 * b0 + 2 * b1, 2
            else:
                start, n = 4 * b0, 4
            rdma = pltpu.make_async_remote_copy(
                src_ref=w_refs[p].at[pl.ds(start, n)],
                dst_ref=w_refs[p].at[pl.ds(start, n)],
                send_sem=send_sems.at[p, 3 + u],
                recv_sem=recv_sems.at[p, 3 + u],
                device_id=(partner_ax[ORDERS[p][2 - u]],),
                device_id_type=pl.DeviceIdType.MESH)
            rdma.start()
            return rdma

        copies = []
        for b in range(B):
            for g in range(2):
                h = 2 * me + g
                copies.append(pltpu.make_async_copy(
                    k_hbm.at[b, :, h, :], kv_ref.at[0, b, g], copy_sem))
                copies.append(pltpu.make_async_copy(
                    v_hbm.at[b, :, h, :], kv_ref.at[1, b, g], copy_sem))
        for cp in copies:
            cp.start()

        q_ref[...] = jnp.dot(x_ref[...].reshape(B * SQ, D), wq_ref[...],
                             preferred_element_type=F32).astype(BF)
        for cp in copies:
            cp.wait()

        border = (2 * (1 - xb) + (1 - yb), 2 * (1 - xb) + yb,
                  2 * xb + (1 - yb), 2 * xb + yb)
        rs_pending = [None, None, None]
        for bi in range(4):
            bb = border[bi]
            xB = lax.div(bb, 2)
            yB = lax.rem(bb, 2)
            for g in range(2):
                k = kv_ref[0, bb, g].astype(BF)
                v = kv_ref[1, bb, g].astype(BF)
                for hh in range(4):
                    h = 4 * g + hh
                    q = q_ref[pl.ds(bb * SQ, SQ), pl.ds(h * DH, DH)]
                    sc = lax.dot_general(
                        q, k, (((1,), (1,)), ((), ())),
                        preferred_element_type=F32) * SCALE
                    p_ = jnp.exp(sc)
                    l = jnp.sum(p_, axis=1, keepdims=True)
                    o = jnp.dot(p_.astype(BF), v,
                                preferred_element_type=F32) / l
                    a_ref[pl.ds(bb * SQ, SQ), pl.ds(h * DH, DH)] = (
                        o.astype(BF))
            pb = jnp.dot(a_ref[pl.ds(bb * SQ, SQ), :], wo_ref[...],
                         preferred_element_type=F32).astype(BF)
            for p in range(3):
                c0, c1 = COLS[p]
                for j in range(2):
                    bits = (xB, yB, j)
                    o = ORDERS[p]
                    slot = 4 * bits[o[0]] + 2 * bits[o[1]] + bits[o[2]]
                    w_refs[p][pl.ds(slot, 1)] = (
                        pb[j * 128:(j + 1) * 128, c0:c1].reshape(1, 128, -1))
            if bi == 1:
                bsem = pltpu.get_barrier_semaphore()
                for peer in (px, py, pz):
                    pl.semaphore_signal(bsem, inc=1, device_id=(peer,),
                                        device_id_type=pl.DeviceIdType.MESH)
                pl.semaphore_wait(bsem, 3)
                rs_pending[0] = rs_rdma(0, 0)
                rs_pending[0][0].start()
            elif bi == 2:
                b00, b01, _ = geom[0]
                rs_pending[0][0].wait()
                pair = 4 * b00 + 2 * (1 - b01)
                w0_ref[pl.ds(pair, 2)] = (
                    w0_ref[pl.ds(pair, 2)]
                    + r0_ref[pl.ds(2 * (1 - b01), 2)])
                rs_pending[0] = rs_rdma(0, 1)
                rs_pending[0][0].start()
                rs_pending[1] = rs_rdma(1, 0)
                rs_pending[1][0].start()
            elif bi == 3:
                b00, b01, _ = geom[0]
                pair = 4 * b00 + 2 * b01
                w0_ref[pl.ds(pair, 2)] = (
                    w0_ref[pl.ds(pair, 2)] + r0_ref[pl.ds(2 * b01, 2)])
                rs_finish(0, 1, rs_pending[0])
                rs_pending[0] = rs_rdma(0, 2)
                rs_pending[0][0].start()
                rs_finish(1, 0, rs_pending[1])
                rs_pending[1] = rs_rdma(1, 1)
                rs_pending[1][0].start()

        rs_pending[2] = rs_rdma(2, 0)
        rs_pending[2][0].start()
        rs_finish(0, 2, rs_pending[0])
        ag0 = ag_rdma(0, 0)
        rs_finish(1, 1, rs_pending[1])
        rs_pending[1] = rs_rdma(1, 2)
        rs_pending[1][0].start()
        rs_finish(2, 0, rs_pending[2])
        rs_pending[2] = rs_rdma(2, 1)
        rs_pending[2][0].start()
        ag0.wait()
        ag0 = ag_rdma(0, 1)
        rs_finish(1, 2, rs_pending[1])
        ag1 = ag_rdma(1, 0)
        rs_finish(2, 1, rs_pending[2])
        rs_pending[2] = rs_rdma(2, 2)
        rs_pending[2][0].start()
        ag0.wait()
        ag0 = ag_rdma(0, 2)
        ag1.wait()
        ag1 = ag_rdma(1, 1)
        rs_finish(2, 2, rs_pending[2])
        ag2 = ag_rdma(2, 0)
        ag0.wait()
        ag1.wait()
        ag1 = ag_rdma(1, 2)
        ag2.wait()
        ag2 = ag_rdma(2, 1)
        ag1.wait()
        ag2.wait()
        ag2 = ag_rdma(2, 2)
        ag2.wait()

        def slot_of(p, c):
            bits = (c >> 2, (c >> 1) & 1, c & 1)
            o = ORDERS[p]
            return 4 * bits[o[0]] + 2 * bits[o[1]] + bits[o[2]]

        for p in range(3):
            c0, c1 = COLS[p]
            for c in range(N_DEV):
                out_ref[c // 2,
                        (c % 2) * 128:(c % 2) * 128 + 128,
                        c0:c1] = w_refs[p][slot_of(p, c)].astype(F32)

    return pl.pallas_call(
        body,
        out_shape=jax.ShapeDtypeStruct((B, SQ, D), F32),
        in_specs=[
            pl.BlockSpec(memory_space=pltpu.VMEM),
            pl.BlockSpec(memory_space=pltpu.VMEM),
            pl.BlockSpec(memory_space=pltpu.VMEM),
            pl.BlockSpec(memory_space=pltpu.MemorySpace.HBM),
            pl.BlockSpec(memory_space=pltpu.MemorySpace.HBM),
        ],
        out_specs=pl.BlockSpec(memory_space=pltpu.VMEM),
        scratch_shapes=[
            pltpu.VMEM((N_DEV, 128, 384), BF),
            pltpu.VMEM((N_DEV, 128, 384), BF),
            pltpu.VMEM((N_DEV, 128, 256), BF),
            pltpu.VMEM((7, 128, 384), BF),
            pltpu.VMEM((7, 128, 384), BF),
            pltpu.VMEM((7, 128, 256), BF),
            pltpu.VMEM((B * SQ, D), BF),
            pltpu.VMEM((2, B, 2, SKV, DH), F32),
            pltpu.SemaphoreType.DMA,
            pltpu.SemaphoreType.DMA((3, 6)),
            pltpu.SemaphoreType.DMA((3, 6)),
        ],
        compiler_params=pltpu.CompilerParams(
            collective_id=0, vmem_limit_bytes=63 * 1024 * 1024),
    )(x.astype(BF), Wq.astype(BF), Wo.astype(BF), K_ext, V_ext)


# device time: 54372 ns/iter; 1.4013x vs baseline; 1.4013x over previous
import jax
import jax.numpy as jnp
from jax import lax
from jax.experimental import pallas as pl
from jax.experimental.pallas import tpu as pltpu

N_DEV = 8
B, SQ, D = 4, 256, 1024
HQ_LOC, DH = 8, 128
SKV = 1024
SCALE = 0.08838834764831843
BF = jnp.bfloat16
F32 = jnp.float32

COLS = ((0, 384), (384, 768), (768, 1024))
ORDERS = ((0, 1, 2), (1, 2, 0), (2, 0, 1))


def kernel(x, Wq, Wo, K_ext, V_ext):
    def body(x_ref, wq_ref, wo_ref, k_hbm, v_hbm, out_ref,
             w0_ref, w1_ref, w2_ref, r0_ref, r1_ref, r2_ref,
             q_ref, kv_ref, copy_sem, send_sems, recv_sems):
        a_ref = q_ref
        me = lax.axis_index("i")
        s = lax.rem(me, 4)
        z = lax.div(me, 4)
        xb = lax.rem(s + lax.div(s, 2), 2)
        yb = lax.div(s, 2)
        px = z * 4 + s + 1 - 2 * lax.rem(s, 2)
        py = z * 4 + 3 - s
        pz = (1 - z) * 4 + s
        partner_ax = (px, py, pz)
        mybit = (xb, yb, z)
        w_refs = (w0_ref, w1_ref, w2_ref)
        r_refs = (r0_ref, r1_ref, r2_ref)

        geom = []
        for p in range(3):
            o = ORDERS[p]
            geom.append((mybit[o[0]], mybit[o[1]], mybit[o[2]]))

        RSTART = (0, 4, 6)

        def rs_rdma(p, t):
            b0, b1, b2 = geom[p]
            if t == 0:
                keep, send, n = 4 * b0, 4 * (1 - b0), 4
            elif t == 1:
                keep, send, n = 4 * b0 + 2 * b1, 4 * b0 + 2 * (1 - b1), 2
            else:
                keep = 4 * b0 + 2 * b1 + b2
                send, n = 4 * b0 + 2 * b1 + (1 - b2), 1
            rdma = pltpu.make_async_remote_copy(
                src_ref=w_refs[p].at[pl.ds(send, n)],
                dst_ref=r_refs[p].at[pl.ds(RSTART[t], n)],
                send_sem=send_sems.at[p, t],
                recv_sem=recv_sems.at[p, t],
                device_id=(partner_ax[ORDERS[p][t]],),
                device_id_type=pl.DeviceIdType.MESH)
            return rdma, keep, n

        def rs_finish(p, t, pending):
            rdma, keep, n = pending
            rdma.wait()
            w_refs[p][pl.ds(keep, n)] = (
                w_refs[p][pl.ds(keep, n)]
                + r_refs[p][pl.ds(RSTART[t], n)])

        def ag_rdma(p, u):
            b0, b1, b2 = geom[p]
            if u == 0:
                start, n = 4 * b0 + 2 * b1 + b2, 1
            elif u == 1:
                start, n = 4 * b0 + 2 * b1, 2
            else:
                start, n = 4 * b0, 4
            rdma = pltpu.make_async_remote_copy(
                src_ref=w_refs[p].at[pl.ds(start, n)],
                dst_ref=w_refs[p].at[pl.ds(start, n)],
                send_sem=send_sems.at[p, 3 + u],
                recv_sem=recv_sems.at[p, 3 + u],
                device_id=(partner_ax[ORDERS[p][2 - u]],),
                device_id_type=pl.DeviceIdType.MESH)
            rdma.start()
            return rdma

        copies = []
        for b in range(B):
            for g in range(2):
                h = 2 * me + g
                copies.append(pltpu.make_async_copy(
                    k_hbm.at[b, :, h, :], kv_ref.at[0, b, g], copy_sem))
                copies.append(pltpu.make_async_copy(
                    v_hbm.at[b, :, h, :], kv_ref.at[1, b, g], copy_sem))
        for cp in copies:
            cp.start()

        bsem = pltpu.get_barrier_semaphore()
        for peer in (px, py, pz):
            pl.semaphore_signal(bsem, inc=1, device_id=(peer,),
                                device_id_type=pl.DeviceIdType.MESH)
        pl.semaphore_wait(bsem, 3)

        q_ref[...] = jnp.dot(x_ref[...].reshape(B * SQ, D).astype(BF),
                             wq_ref[...].astype(BF),
                             preferred_element_type=F32).astype(BF)
        for cp in copies:
            cp.wait()

        border = (2 * (1 - xb) + (1 - yb), 2 * (1 - xb) + yb,
                  2 * xb + (1 - yb), 2 * xb + yb)
        rs_pending = [None, None, None]
        for bi in range(4):
            bb = border[bi]
            xB = lax.div(bb, 2)
            yB = lax.rem(bb, 2)
            for g in range(2):
                k = kv_ref[0, bb, g].astype(BF)
                v = kv_ref[1, bb, g].astype(BF)
                for hh in range(4):
                    h = 4 * g + hh
                    q = q_ref[pl.ds(bb * SQ, SQ), pl.ds(h * DH, DH)]
                    sc = lax.dot_general(
                        q, k, (((1,), (1,)), ((), ())),
                        preferred_element_type=F32) * SCALE
                    p_ = jnp.exp(sc)
                    l = jnp.sum(p_, axis=1, keepdims=True)
                    o = jnp.dot(p_.astype(BF), v,
                                preferred_element_type=F32) / l
                    a_ref[pl.ds(bb * SQ, SQ), pl.ds(h * DH, DH)] = (
                        o.astype(BF))
            pb = jnp.dot(a_ref[pl.ds(bb * SQ, SQ), :], wo_ref[...],
                         preferred_element_type=F32).astype(BF)
            for p in range(3):
                c0, c1 = COLS[p]
                for j in range(2):
                    bits = (xB, yB, j)
                    o = ORDERS[p]
                    slot = 4 * bits[o[0]] + 2 * bits[o[1]] + bits[o[2]]
                    w_refs[p][pl.ds(slot, 1)] = (
                        pb[j * 128:(j + 1) * 128, c0:c1].reshape(1, 128, -1))
            if bi == 1:
                rs_pending[0] = rs_rdma(0, 0)
                rs_pending[0][0].start()
            elif bi == 2:
                rs_pending[1] = rs_rdma(1, 0)
                rs_pending[1][0].start()
            elif bi == 3:
                rs_pending[2] = rs_rdma(2, 0)
                rs_pending[2][0].start()

        for p in range(3):
            rs_finish(p, 0, rs_pending[p])
            rs_pending[p] = rs_rdma(p, 1)
            rs_pending[p][0].start()

        mids = []
        for p in range(3):
            rs_finish(p, 1, rs_pending[p])
            b0, b1, _ = geom[p]
            g2 = 4 * b0 + 2 * b1
            rdma = pltpu.make_async_remote_copy(
                src_ref=w_refs[p].at[pl.ds(g2, 2)],
                dst_ref=r_refs[p].at[pl.ds(6, 2)],
                send_sem=send_sems.at[p, 2],
                recv_sem=recv_sems.at[p, 2],
                device_id=(partner_ax[ORDERS[p][2]],),
                device_id_type=pl.DeviceIdType.MESH)
            rdma.start()
            mids.append((rdma, g2))
        def store_pair(p, gb, b1v):
            if p == 0:
                out_ref[pl.ds(2 * gb + b1v, 1), :, 0:384] = (
                    w0_ref[pl.ds(4 * gb + 2 * b1v, 2)].astype(F32)
                    .reshape(1, 256, 384))
            elif p == 1:
                for xc in range(2):
                    out_ref[pl.ds(2 * xc + gb, 1),
                            pl.ds(b1v * 128, 128), 384:768] = (
                        w1_ref[pl.ds(4 * gb + 2 * b1v + xc, 1)].astype(F32))
            else:
                for yc in range(2):
                    out_ref[pl.ds(2 * b1v + yc, 1),
                            pl.ds(gb * 128, 128), 768:1024] = (
                        w2_ref[pl.ds(4 * gb + 2 * b1v + yc, 1)].astype(F32))

        ag1s = []
        for p in range(3):
            rdma, g2 = mids[p]
            rdma.wait()
            w_refs[p][pl.ds(g2, 2)] = (
                w_refs[p][pl.ds(g2, 2)] + r_refs[p][pl.ds(6, 2)])
            ag1s.append(ag_rdma(p, 1))
        for p in range(3):
            store_pair(p, geom[p][0], geom[p][1])
        ag2s = []
        for p in range(3):
            ag1s[p].wait()
            ag2s.append(ag_rdma(p, 2))
        for p in range(3):
            store_pair(p, geom[p][0], 1 - geom[p][1])

        def store_group(p, gb):
            base = 4 * gb
            if p == 0:
                out_ref[pl.ds(2 * gb, 2), :, 0:384] = (
                    w0_ref[pl.ds(base, 4)].astype(F32).reshape(2, 256, 384))
            elif p == 1:
                for zc in range(2):
                    for xc in range(2):
                        out_ref[pl.ds(2 * xc + gb, 1),
                                zc * 128:(zc + 1) * 128, 384:768] = (
                            w1_ref[pl.ds(base + 2 * zc + xc, 1)]
                            .astype(F32))
            else:
                for b in range(4):
                    out_ref[b, pl.ds(gb * 128, 128), 768:1024] = (
                        w2_ref[pl.ds(base + b, 1)].astype(F32)
                        .reshape(128, 256))

        for p in range(3):
            ag2s[p].wait()
            store_group(p, 1 - geom[p][0])

    return pl.pallas_call(
        body,
        out_shape=jax.ShapeDtypeStruct((B, SQ, D), F32),
        in_specs=[
            pl.BlockSpec(memory_space=pltpu.VMEM),
            pl.BlockSpec(memory_space=pltpu.VMEM),
            pl.BlockSpec(memory_space=pltpu.VMEM),
            pl.BlockSpec(memory_space=pltpu.MemorySpace.HBM),
            pl.BlockSpec(memory_space=pltpu.MemorySpace.HBM),
        ],
        out_specs=pl.BlockSpec(memory_space=pltpu.VMEM),
        scratch_shapes=[
            pltpu.VMEM((N_DEV, 128, 384), BF),
            pltpu.VMEM((N_DEV, 128, 384), BF),
            pltpu.VMEM((N_DEV, 128, 256), BF),
            pltpu.VMEM((8, 128, 384), BF),
            pltpu.VMEM((8, 128, 384), BF),
            pltpu.VMEM((8, 128, 256), BF),
            pltpu.VMEM((B * SQ, D), BF),
            pltpu.VMEM((2, B, 2, SKV, DH), F32),
            pltpu.SemaphoreType.DMA,
            pltpu.SemaphoreType.DMA((3, 6)),
            pltpu.SemaphoreType.DMA((3, 6)),
        ],
        compiler_params=pltpu.CompilerParams(
            collective_id=0, vmem_limit_bytes=63 * 1024 * 1024),
    )(x, Wq, Wo.astype(BF), K_ext, V_ext)
